# initial kernel scaffold (unmeasured)
import jax
import jax.numpy as jnp
from jax import lax
from jax.experimental import pallas as pl
from jax.experimental.pallas import tpu as pltpu


def kernel(
    x,
):
    def body(*refs):
        pass

    out_shape = jax.ShapeDtypeStruct(..., jnp.float32)
    return pl.pallas_call(body, out_shape=out_shape)(...)


# baseline (device time: 158157 ns/iter reference)
import jax
import jax.numpy as jnp
from jax import lax
from jax.experimental import pallas as pl
from jax.experimental.pallas import tpu as pltpu

N_DEV = 4


def kernel(x):
    m, n = x.shape
    chunk = m // N_DEV

    def body(
        x_ref,
        out_ref,
        xb_ref,
        acc_ref,
        rs_recv_ref,
        send_sems,
        rs_recv_sems,
        ag_recv_sems,
    ):
        my = lax.axis_index("i")
        left = (my - 1) % N_DEV
        right = (my + 1) % N_DEV

        barrier_sem = pltpu.get_barrier_semaphore()
        for nbr in (left, right):
            pl.semaphore_signal(
                barrier_sem,
                inc=1,
                device_id=(nbr,),
                device_id_type=pl.DeviceIdType.MESH,
            )
        pl.semaphore_wait(barrier_sem, 2)

        xb_ref[...] = x_ref[...].astype(jnp.bfloat16)

        for s in range(N_DEV - 1):
            send_idx = (my - s) % N_DEV
            if s == 0:
                src = xb_ref.at[pl.ds(send_idx * chunk, chunk), :]
            else:
                acc_ref[...] = (
                    rs_recv_ref[s - 1]
                    + xb_ref[pl.ds(send_idx * chunk, chunk), :]
                )
                src = acc_ref
            rdma = pltpu.make_async_remote_copy(
                src_ref=src,
                dst_ref=rs_recv_ref.at[s],
                send_sem=send_sems.at[s],
                recv_sem=rs_recv_sems.at[s],
                device_id=(right,),
                device_id_type=pl.DeviceIdType.MESH,
            )
            rdma.start()
            rdma.wait()

        red_idx = (my + 1) % N_DEV
        out_ref[pl.ds(red_idx * chunk, chunk), :] = (
            rs_recv_ref[N_DEV - 2]
            + xb_ref[pl.ds(red_idx * chunk, chunk), :]
        )

        for t in range(N_DEV - 1):
            idx = (my + 1 - t) % N_DEV
            rdma = pltpu.make_async_remote_copy(
                src_ref=out_ref.at[pl.ds(idx * chunk, chunk), :],
                dst_ref=out_ref.at[pl.ds(idx * chunk, chunk), :],
                send_sem=send_sems.at[N_DEV - 1 + t],
                recv_sem=ag_recv_sems.at[t],
                device_id=(right,),
                device_id_type=pl.DeviceIdType.MESH,
            )
            rdma.start()
            rdma.wait()

    return pl.pallas_call(
        body,
        out_shape=jax.ShapeDtypeStruct((m, n), jnp.bfloat16),
        in_specs=[pl.BlockSpec(memory_space=pltpu.VMEM)],
        out_specs=pl.BlockSpec(memory_space=pltpu.VMEM),
        scratch_shapes=[
            pltpu.VMEM((m, n), jnp.bfloat16),
            pltpu.VMEM((chunk, n), jnp.bfloat16),
            pltpu.VMEM((N_DEV - 1, chunk, n), jnp.bfloat16),
            pltpu.SemaphoreType.DMA((2 * (N_DEV - 1),)),
            pltpu.SemaphoreType.DMA((N_DEV - 1,)),
            pltpu.SemaphoreType.DMA((N_DEV - 1,)),
        ],
        compiler_params=pltpu.CompilerParams(collective_id=0),
    )(x)


# device time: 90808 ns/iter; 1.7417x vs baseline; 1.7417x over previous
import jax
import jax.numpy as jnp
from jax import lax
from jax.experimental import pallas as pl
from jax.experimental.pallas import tpu as pltpu

N_DEV = 4
N_HOP = N_DEV - 1


def kernel(x):
    m, n = x.shape
    chunk = m // N_DEV
    half = n // 2

    def body(
        x_ref,
        out_ref,
        xb_ref,
        acc_ref,
        rs_recv_ref,
        send_sems,
        rs_recv_sems,
        ag_recv_sems,
    ):
        my = lax.axis_index("i")
        left = (my - 1) % N_DEV
        right = (my + 1) % N_DEV

        xb_ref[...] = x_ref[...].astype(jnp.bfloat16)

        barrier_sem = pltpu.get_barrier_semaphore()
        for nbr in (left, right):
            pl.semaphore_signal(
                barrier_sem,
                inc=1,
                device_id=(nbr,),
                device_id_type=pl.DeviceIdType.MESH,
            )
        pl.semaphore_wait(barrier_sem, 2)

        dirs = ((right, -1, 0), (left, +1, half))

        for s in range(N_HOP):
            rdmas = []
            for d, (dst, sign, col) in enumerate(dirs):
                send_idx = (my + sign * s) % N_DEV
                if s == 0:
                    src = xb_ref.at[
                        pl.ds(send_idx * chunk, chunk), pl.ds(col, half)
                    ]
                else:
                    acc_ref[d] = (
                        rs_recv_ref[d, s - 1]
                        + xb_ref[pl.ds(send_idx * chunk, chunk), pl.ds(col, half)]
                    )
                    src = acc_ref.at[d]
                rdma = pltpu.make_async_remote_copy(
                    src_ref=src,
                    dst_ref=rs_recv_ref.at[d, s],
                    send_sem=send_sems.at[d * N_HOP + s],
                    recv_sem=rs_recv_sems.at[d * N_HOP + s],
                    device_id=(dst,),
                    device_id_type=pl.DeviceIdType.MESH,
                )
                rdma.start()
                rdmas.append(rdma)
            for rdma in rdmas:
                rdma.wait()

        for d, (dst, sign, col) in enumerate(dirs):
            red_idx = (my - sign) % N_DEV
            out_ref[pl.ds(red_idx * chunk, chunk), pl.ds(col, half)] = (
                rs_recv_ref[d, N_HOP - 1]
                + xb_ref[pl.ds(red_idx * chunk, chunk), pl.ds(col, half)]
            )

        for t in range(N_HOP):
            rdmas = []
            for d, (dst, sign, col) in enumerate(dirs):
                idx = (my - sign + sign * t) % N_DEV
                rdma = pltpu.make_async_remote_copy(
                    src_ref=out_ref.at[pl.ds(idx * chunk, chunk), pl.ds(col, half)],
                    dst_ref=out_ref.at[pl.ds(idx * chunk, chunk), pl.ds(col, half)],
                    send_sem=send_sems.at[2 * N_HOP + d * N_HOP + t],
                    recv_sem=ag_recv_sems.at[d * N_HOP + t],
                    device_id=(dst,),
                    device_id_type=pl.DeviceIdType.MESH,
                )
                rdma.start()
                rdmas.append(rdma)
            for rdma in rdmas:
                rdma.wait()

    return pl.pallas_call(
        body,
        out_shape=jax.ShapeDtypeStruct((m, n), jnp.bfloat16),
        in_specs=[pl.BlockSpec(memory_space=pltpu.VMEM)],
        out_specs=pl.BlockSpec(memory_space=pltpu.VMEM),
        scratch_shapes=[
            pltpu.VMEM((m, n), jnp.bfloat16),
            pltpu.VMEM((2, chunk, half), jnp.bfloat16),
            pltpu.VMEM((2, N_HOP, chunk, half), jnp.bfloat16),
            pltpu.SemaphoreType.DMA((4 * N_HOP,)),
            pltpu.SemaphoreType.DMA((2 * N_HOP,)),
            pltpu.SemaphoreType.DMA((2 * N_HOP,)),
        ],
        compiler_params=pltpu.CompilerParams(collective_id=0),
    )(x)


# device time: 81857 ns/iter; 1.9321x vs baseline; 1.1093x over previous
import jax
import jax.numpy as jnp
from jax import lax
from jax.experimental import pallas as pl
from jax.experimental.pallas import tpu as pltpu

N_DEV = 4
N_HOP = N_DEV - 1
P = 2


def kernel(x):
    m, n = x.shape
    chunk = m // N_DEV
    half = n // 2
    sub = chunk // P

    def body(
        x_ref,
        out_ref,
        stage0_ref,
        acc_ref,
        rs_recv_ref,
        rs_send_sems,
        rs_recv_sems,
        ag_send_sems,
        ag_recv_sems,
    ):
        my = lax.axis_index("i")
        left = (my - 1) % N_DEV
        right = (my + 1) % N_DEV

        def xb(idx, row, col):
            return x_ref[
                pl.ds(idx * chunk + row, sub), pl.ds(col, half)
            ].astype(jnp.bfloat16)

        stage0_ref[...] = x_ref[pl.ds(my * chunk, chunk), :].astype(jnp.bfloat16)

        barrier_sem = pltpu.get_barrier_semaphore()
        for nbr in (left, right):
            pl.semaphore_signal(
                barrier_sem,
                inc=1,
                device_id=(nbr,),
                device_id_type=pl.DeviceIdType.MESH,
            )
        pl.semaphore_wait(barrier_sem, 2)

        dirs = ((right, -1, 0), (left, +1, half))

        all_sends = []
        rs_rdmas = [[[None] * P for _ in range(N_HOP)] for _ in range(2)]
        ag_rdmas = [[[None] * P for _ in range(N_HOP)] for _ in range(2)]

        def start_rs(d, s, p):
            dst, sign, col = dirs[d]
            row = p * sub
            if s == 0:
                src = stage0_ref.at[pl.ds(row, sub), pl.ds(col, half)]
            else:
                send_idx = (my + sign * s) % N_DEV
                acc_ref[d, s - 1, pl.ds(row, sub), :] = (
                    rs_recv_ref[d, s - 1, pl.ds(row, sub), :]
                    + xb(send_idx, row, col)
                )
                src = acc_ref.at[d, s - 1, pl.ds(row, sub), :]
            rdma = pltpu.make_async_remote_copy(
                src_ref=src,
                dst_ref=rs_recv_ref.at[d, s, pl.ds(row, sub), :],
                send_sem=rs_send_sems.at[d, s, p],
                recv_sem=rs_recv_sems.at[d, s, p],
                device_id=(dst,),
                device_id_type=pl.DeviceIdType.MESH,
            )
            rdma.start()
            rs_rdmas[d][s][p] = rdma
            all_sends.append(rdma)

        def start_ag(d, t, p):
            dst, sign, col = dirs[d]
            row = p * sub
            idx = (my - sign + sign * t) % N_DEV
            sl = (pl.ds(idx * chunk + row, sub), pl.ds(col, half))
            rdma = pltpu.make_async_remote_copy(
                src_ref=out_ref.at[sl],
                dst_ref=out_ref.at[sl],
                send_sem=ag_send_sems.at[d, t, p],
                recv_sem=ag_recv_sems.at[d, t, p],
                device_id=(dst,),
                device_id_type=pl.DeviceIdType.MESH,
            )
            rdma.start()
            ag_rdmas[d][t][p] = rdma
            all_sends.append(rdma)

        for p in range(P):
            for d in range(2):
                start_rs(d, 0, p)
        for s in range(1, N_HOP):
            for p in range(P):
                for d in range(2):
                    rs_rdmas[d][s - 1][p].wait_recv()
                    start_rs(d, s, p)

        for p in range(P):
            for d in range(2):
                dst, sign, col = dirs[d]
                row = p * sub
                red_idx = (my - sign) % N_DEV
                rs_rdmas[d][N_HOP - 1][p].wait_recv()
                out_ref[pl.ds(red_idx * chunk + row, sub), pl.ds(col, half)] = (
                    rs_recv_ref[d, N_HOP - 1, pl.ds(row, sub), :]
                    + xb(red_idx, row, col)
                )
                start_ag(d, 0, p)

        for t in range(1, N_HOP):
            for p in range(P):
                for d in range(2):
                    ag_rdmas[d][t - 1][p].wait_recv()
                    start_ag(d, t, p)

        for p in range(P):
            for d in range(2):
                ag_rdmas[d][N_HOP - 1][p].wait_recv()

        for rdma in all_sends:
            rdma.wait_send()

    return pl.pallas_call(
        body,
        out_shape=jax.ShapeDtypeStruct((m, n), jnp.bfloat16),
        in_specs=[pl.BlockSpec(memory_space=pltpu.VMEM)],
        out_specs=pl.BlockSpec(memory_space=pltpu.VMEM),
        scratch_shapes=[
            pltpu.VMEM((chunk, n), jnp.bfloat16),
            pltpu.VMEM((2, N_HOP - 1, chunk, half), jnp.bfloat16),
            pltpu.VMEM((2, N_HOP, chunk, half), jnp.bfloat16),
            pltpu.SemaphoreType.DMA((2, N_HOP, P)),
            pltpu.SemaphoreType.DMA((2, N_HOP, P)),
            pltpu.SemaphoreType.DMA((2, N_HOP, P)),
            pltpu.SemaphoreType.DMA((2, N_HOP, P)),
        ],
        compiler_params=pltpu.CompilerParams(collective_id=0),
    )(x)


# device time: 81215 ns/iter; 1.9474x vs baseline; 1.0079x over previous
import jax
import jax.numpy as jnp
from jax import lax
from jax.experimental import pallas as pl
from jax.experimental.pallas import tpu as pltpu

N_DEV = 4
N_HOP = N_DEV - 1
P = 4


def kernel(x):
    m, n = x.shape
    chunk = m // N_DEV
    half = n // 2
    sub = chunk // P

    def body(
        x_ref,
        out_ref,
        stage0_ref,
        acc_ref,
        rs_recv_ref,
        rs_send_sems,
        rs_recv_sems,
        ag_send_sems,
        ag_recv_sems,
    ):
        my = lax.axis_index("i")
        left = (my - 1) % N_DEV
        right = (my + 1) % N_DEV

        def xb(idx, row, col):
            return x_ref[
                pl.ds(idx * chunk + row, sub), pl.ds(col, half)
            ].astype(jnp.bfloat16)

        stage0_ref[...] = x_ref[pl.ds(my * chunk, chunk), :].astype(jnp.bfloat16)

        barrier_sem = pltpu.get_barrier_semaphore()
        for nbr in (left, right):
            pl.semaphore_signal(
                barrier_sem,
                inc=1,
                device_id=(nbr,),
                device_id_type=pl.DeviceIdType.MESH,
            )
        pl.semaphore_wait(barrier_sem, 2)

        dirs = ((right, -1, 0), (left, +1, half))

        all_sends = []
        rs_rdmas = [[[None] * P for _ in range(N_HOP)] for _ in range(2)]
        ag_rdmas = [[[None] * P for _ in range(N_HOP)] for _ in range(2)]

        def start_rs(d, s, p):
            dst, sign, col = dirs[d]
            row = p * sub
            if s == 0:
                src = stage0_ref.at[pl.ds(row, sub), pl.ds(col, half)]
            else:
                send_idx = (my + sign * s) % N_DEV
                acc_ref[d, s - 1, pl.ds(row, sub), :] = (
                    rs_recv_ref[d, s - 1, pl.ds(row, sub), :]
                    + xb(send_idx, row, col)
                )
                src = acc_ref.at[d, s - 1, pl.ds(row, sub), :]
            rdma = pltpu.make_async_remote_copy(
                src_ref=src,
                dst_ref=rs_recv_ref.at[d, s, pl.ds(row, sub), :],
                send_sem=rs_send_sems.at[d, s, p],
                recv_sem=rs_recv_sems.at[d, s, p],
                device_id=(dst,),
                device_id_type=pl.DeviceIdType.MESH,
            )
            rdma.start()
            rs_rdmas[d][s][p] = rdma
            all_sends.append(rdma)

        def start_ag(d, t, p):
            dst, sign, col = dirs[d]
            row = p * sub
            idx = (my - sign + sign * t) % N_DEV
            sl = (pl.ds(idx * chunk + row, sub), pl.ds(col, half))
            rdma = pltpu.make_async_remote_copy(
                src_ref=out_ref.at[sl],
                dst_ref=out_ref.at[sl],
                send_sem=ag_send_sems.at[d, t, p],
                recv_sem=ag_recv_sems.at[d, t, p],
                device_id=(dst,),
                device_id_type=pl.DeviceIdType.MESH,
            )
            rdma.start()
            ag_rdmas[d][t][p] = rdma
            all_sends.append(rdma)

        for p in range(P):
            for d in range(2):
                start_rs(d, 0, p)
        for s in range(1, N_HOP):
            for p in range(P):
                for d in range(2):
                    rs_rdmas[d][s - 1][p].wait_recv()
                    start_rs(d, s, p)

        for p in range(P):
            for d in range(2):
                dst, sign, col = dirs[d]
                row = p * sub
                red_idx = (my - sign) % N_DEV
                rs_rdmas[d][N_HOP - 1][p].wait_recv()
                out_ref[pl.ds(red_idx * chunk + row, sub), pl.ds(col, half)] = (
                    rs_recv_ref[d, N_HOP - 1, pl.ds(row, sub), :]
                    + xb(red_idx, row, col)
                )
                start_ag(d, 0, p)

        for t in range(1, N_HOP):
            for p in range(P):
                for d in range(2):
                    ag_rdmas[d][t - 1][p].wait_recv()
                    start_ag(d, t, p)

        for p in range(P):
            for d in range(2):
                ag_rdmas[d][N_HOP - 1][p].wait_recv()

        for rdma in all_sends:
            rdma.wait_send()

    return pl.pallas_call(
        body,
        out_shape=jax.ShapeDtypeStruct((m, n), jnp.bfloat16),
        in_specs=[pl.BlockSpec(memory_space=pltpu.VMEM)],
        out_specs=pl.BlockSpec(memory_space=pltpu.VMEM),
        scratch_shapes=[
            pltpu.VMEM((chunk, n), jnp.bfloat16),
            pltpu.VMEM((2, N_HOP - 1, chunk, half), jnp.bfloat16),
            pltpu.VMEM((2, N_HOP, chunk, half), jnp.bfloat16),
            pltpu.SemaphoreType.DMA((2, N_HOP, P)),
            pltpu.SemaphoreType.DMA((2, N_HOP, P)),
            pltpu.SemaphoreType.DMA((2, N_HOP, P)),
            pltpu.SemaphoreType.DMA((2, N_HOP, P)),
        ],
        compiler_params=pltpu.CompilerParams(collective_id=0),
    )(x)


# device time: 77834 ns/iter; 2.0320x vs baseline; 1.0434x over previous
import jax
import jax.numpy as jnp
from jax import lax
from jax.experimental import pallas as pl
from jax.experimental.pallas import tpu as pltpu

N_DEV = 4
N_HOP = N_DEV - 1
P = 4


def kernel(x):
    m, n = x.shape
    chunk = m // N_DEV
    half = n // 2
    sub = chunk // P

    def body(
        x_ref,
        out_ref,
        xv_ref,
        stage0_ref,
        acc_ref,
        rs_recv_ref,
        load_sems,
        rs_send_sems,
        rs_recv_sems,
        ag_send_sems,
        ag_recv_sems,
    ):
        my = lax.axis_index("i")
        left = (my - 1) % N_DEV
        right = (my + 1) % N_DEV

        load_copies = {}
        for o in (0, 1, 3, 2):
            idx = (my + o) % N_DEV
            cp = pltpu.make_async_copy(
                x_ref.at[pl.ds(idx * chunk, chunk), :],
                xv_ref.at[pl.ds(idx * chunk, chunk), :],
                load_sems.at[o],
            )
            cp.start()
            load_copies[o] = cp

        waited = set()

        def wait_chunk(o):
            if o not in waited:
                load_copies[o].wait()
                waited.add(o)

        def xb(idx, row, col):
            return xv_ref[
                pl.ds(idx * chunk + row, sub), pl.ds(col, half)
            ].astype(jnp.bfloat16)

        barrier_sem = pltpu.get_barrier_semaphore()
        for nbr in (left, right):
            pl.semaphore_signal(
                barrier_sem,
                inc=1,
                device_id=(nbr,),
                device_id_type=pl.DeviceIdType.MESH,
            )
        pl.semaphore_wait(barrier_sem, 2)

        wait_chunk(0)
        stage0_ref[...] = xv_ref[pl.ds(my * chunk, chunk), :].astype(jnp.bfloat16)

        dirs = ((right, -1, 0), (left, +1, half))

        all_sends = []
        rs_rdmas = [[[None] * P for _ in range(N_HOP)] for _ in range(2)]
        ag_rdmas = [[[None] * P for _ in range(N_HOP)] for _ in range(2)]

        def start_rs(d, s, p):
            dst, sign, col = dirs[d]
            row = p * sub
            if s == 0:
                src = stage0_ref.at[pl.ds(row, sub), pl.ds(col, half)]
            else:
                send_idx = (my + sign * s) % N_DEV
                wait_chunk((sign * s) % N_DEV)
                acc_ref[d, s - 1, pl.ds(row, sub), :] = (
                    rs_recv_ref[d, s - 1, pl.ds(row, sub), :]
                    + xb(send_idx, row, col)
                )
                src = acc_ref.at[d, s - 1, pl.ds(row, sub), :]
            rdma = pltpu.make_async_remote_copy(
                src_ref=src,
                dst_ref=rs_recv_ref.at[d, s, pl.ds(row, sub), :],
                send_sem=rs_send_sems.at[d, s, p],
                recv_sem=rs_recv_sems.at[d, s, p],
                device_id=(dst,),
                device_id_type=pl.DeviceIdType.MESH,
            )
            rdma.start()
            rs_rdmas[d][s][p] = rdma
            all_sends.append(rdma)

        def start_ag(d, t, p):
            dst, sign, col = dirs[d]
            row = p * sub
            idx = (my - sign + sign * t) % N_DEV
            sl = (pl.ds(idx * chunk + row, sub), pl.ds(col, half))
            rdma = pltpu.make_async_remote_copy(
                src_ref=out_ref.at[sl],
                dst_ref=out_ref.at[sl],
                send_sem=ag_send_sems.at[d, t, p],
                recv_sem=ag_recv_sems.at[d, t, p],
                device_id=(dst,),
                device_id_type=pl.DeviceIdType.MESH,
            )
            rdma.start()
            ag_rdmas[d][t][p] = rdma
            all_sends.append(rdma)

        for p in range(P):
            for d in range(2):
                start_rs(d, 0, p)
        for s in range(1, N_HOP):
            for p in range(P):
                for d in range(2):
                    rs_rdmas[d][s - 1][p].wait_recv()
                    start_rs(d, s, p)

        for p in range(P):
            for d in range(2):
                dst, sign, col = dirs[d]
                row = p * sub
                red_idx = (my - sign) % N_DEV
                rs_rdmas[d][N_HOP - 1][p].wait_recv()
                out_ref[pl.ds(red_idx * chunk + row, sub), pl.ds(col, half)] = (
                    rs_recv_ref[d, N_HOP - 1, pl.ds(row, sub), :]
                    + xb(red_idx, row, col)
                )
                start_ag(d, 0, p)

        for t in range(1, N_HOP):
            for p in range(P):
                for d in range(2):
                    ag_rdmas[d][t - 1][p].wait_recv()
                    start_ag(d, t, p)

        for p in range(P):
            for d in range(2):
                ag_rdmas[d][N_HOP - 1][p].wait_recv()

        for rdma in all_sends:
            rdma.wait_send()

    return pl.pallas_call(
        body,
        out_shape=jax.ShapeDtypeStruct((m, n), jnp.bfloat16),
        in_specs=[pl.BlockSpec(memory_space=pl.ANY)],
        out_specs=pl.BlockSpec(memory_space=pltpu.VMEM),
        scratch_shapes=[
            pltpu.VMEM((m, n), jnp.float32),
            pltpu.VMEM((chunk, n), jnp.bfloat16),
            pltpu.VMEM((2, N_HOP - 1, chunk, half), jnp.bfloat16),
            pltpu.VMEM((2, N_HOP, chunk, half), jnp.bfloat16),
            pltpu.SemaphoreType.DMA((N_DEV,)),
            pltpu.SemaphoreType.DMA((2, N_HOP, P)),
            pltpu.SemaphoreType.DMA((2, N_HOP, P)),
            pltpu.SemaphoreType.DMA((2, N_HOP, P)),
            pltpu.SemaphoreType.DMA((2, N_HOP, P)),
        ],
        compiler_params=pltpu.CompilerParams(collective_id=0),
    )(x)
